# baseline (speedup 1.0000x reference)
import functools

import jax
import jax.numpy as jnp
from jax import lax
from jax.experimental import pallas as pl
from jax.experimental.pallas import tpu as pltpu
from jax.experimental.pallas import tpu_sc as plsc

N_NODES = 10000
F_IN = 10000
F_OUT = 5
FP = 8
NC, NS = 2, 16
NW = NC * NS
CHUNK = 2000


def _mm_body(x_ref, w_ref, o_ref):
    o_ref[...] = jnp.dot(x_ref[...], w_ref[...],
                         preferred_element_type=jnp.float32)


def _matmul(x, w):
    m, k = x.shape
    f = w.shape[1]
    bm = 512
    return pl.pallas_call(
        _mm_body,
        grid=(pl.cdiv(m, bm),),
        in_specs=[
            pl.BlockSpec((bm, k), lambda i: (i, 0)),
            pl.BlockSpec((k, f), lambda i: (0, 0)),
        ],
        out_specs=pl.BlockSpec((bm, f), lambda i: (i, 0)),
        out_shape=jax.ShapeDtypeStruct((m, f), jnp.float32),
    )(x, w)


def _hist_body(col_hbm, ones_hbm, zeros_hbm, deg_hbm, colbuf, valbuf, acc):
    cid = lax.axis_index("c")
    sid = lax.axis_index("s")
    wid = sid * NC + cid
    epw = col_hbm.shape[0] // NW

    @pl.when(sid == 0)
    def _():
        pltpu.sync_copy(zeros_hbm, acc)

    pltpu.sync_copy(col_hbm.at[pl.ds(wid * epw, epw)], colbuf)
    pltpu.sync_copy(ones_hbm, valbuf)
    plsc.subcore_barrier()
    pltpu.sync_copy(valbuf, acc.at[colbuf], add=True)
    plsc.subcore_barrier()

    @pl.when(sid == 0)
    def _():
        pltpu.sync_copy(acc, deg_hbm.at[cid])


def _hist(col, ones_c, zeros_n):
    epw = col.shape[0] // NW
    kfn = pl.kernel(
        _hist_body,
        out_type=jax.ShapeDtypeStruct((NC, N_NODES), jnp.float32),
        mesh=plsc.VectorSubcoreMesh(core_axis_name="c", subcore_axis_name="s",
                                    num_cores=NC, num_subcores=NS),
        compiler_params=pltpu.CompilerParams(use_tc_tiling_on_sc=False,
                                             needs_layout_passes=False,
                                             has_side_effects=pltpu.SideEffectType.PURE),
        scratch_types=[
            pltpu.VMEM((epw,), jnp.int32),
            pltpu.VMEM((epw,), jnp.float32),
            pltpu.VMEM_SHARED((N_NODES,), jnp.float32),
        ],
    )
    return kfn(col, ones_c, zeros_n)


def _scale_body(xw_ref, degp_ref, y_ref, dinv_ref):
    deg = degp_ref[0] + degp_ref[1] + 1.0
    dinv = lax.rsqrt(deg)
    dinv_ref[...] = dinv
    y_ref[...] = xw_ref[...] * dinv


def _scale(xw_pad, degp):
    return pl.pallas_call(
        _scale_body,
        out_shape=(
            jax.ShapeDtypeStruct((N_NODES, FP), jnp.float32),
            jax.ShapeDtypeStruct((N_NODES, 1), jnp.float32),
        ),
    )(xw_pad, degp)


def _edge1_body(row_hbm, col_hbm, y_hbm, zeros_hbm, out_hbm,
                rowbuf, colbuf, rows, acc, sem0, sem1):
    cid = lax.axis_index("c")
    sid = lax.axis_index("s")
    wid = sid * NC + cid
    epw = row_hbm.shape[0] // NW
    nchunks = epw // CHUNK
    sems = (sem0, sem1)

    @pl.when(sid == 0)
    def _():
        pltpu.sync_copy(zeros_hbm, acc)

    plsc.subcore_barrier()

    def load_idx(j, b):
        base = wid * epw + j * CHUNK
        pltpu.sync_copy(row_hbm.at[pl.ds(base, CHUNK)], rowbuf.at[b])
        pltpu.sync_copy(col_hbm.at[pl.ds(base, CHUNK)], colbuf.at[b])

    load_idx(0, 0)
    gathers = [pltpu.async_copy(y_hbm.at[rowbuf.at[0]], rows.at[0], sems[0])]
    for j in range(nchunks):
        b = j % 2
        nb = (j + 1) % 2
        if j + 1 < nchunks:
            load_idx(j + 1, nb)
            gathers.append(
                pltpu.async_copy(y_hbm.at[rowbuf.at[nb]], rows.at[nb],
                                 sems[nb]))
        gathers[j].wait()
        pltpu.sync_copy(rows.at[b], acc.at[colbuf.at[b]], add=True)
    plsc.subcore_barrier()

    @pl.when(sid == 0)
    def _():
        pltpu.sync_copy(acc, out_hbm.at[cid])


def _edge1(row, col, y, zeros_n16):
    kfn = pl.kernel(
        _edge1_body,
        out_type=jax.ShapeDtypeStruct((NC, N_NODES, FP), jnp.float32),
        mesh=plsc.VectorSubcoreMesh(core_axis_name="c", subcore_axis_name="s",
                                    num_cores=NC, num_subcores=NS),
        compiler_params=pltpu.CompilerParams(use_tc_tiling_on_sc=False,
                                             needs_layout_passes=False,
                                             has_side_effects=pltpu.SideEffectType.PURE),
        scratch_types=[
            pltpu.VMEM((2, CHUNK), jnp.int32),
            pltpu.VMEM((2, CHUNK), jnp.int32),
            pltpu.VMEM((2, CHUNK, FP), jnp.float32),
            pltpu.VMEM_SHARED((N_NODES, FP), jnp.float32),
            pltpu.SemaphoreType.DMA,
            pltpu.SemaphoreType.DMA,
        ],
    )
    return kfn(row, col, y, zeros_n16)


def _mid_body(p1_ref, y_ref, dinv_ref, b1p_ref, w2p_ref, u_ref):
    s = p1_ref[0] + p1_ref[1] + y_ref[...]
    dinv = dinv_ref[...]
    h = jnp.maximum(s * dinv + b1p_ref[...], 0.0)
    xw2 = jnp.sum(h * w2p_ref[...], axis=1, keepdims=True)
    u_ref[...] = xw2 * dinv


def _mid(p1, y, dinv, b1p, w2p):
    return pl.pallas_call(
        _mid_body,
        out_shape=jax.ShapeDtypeStruct((N_NODES, 1), jnp.float32),
    )(p1, y, dinv, b1p, w2p)


def _edge2_body(row_hbm, col_hbm, u_hbm, zeros_hbm, out_hbm,
                rowbuf, colbuf, vals, u_local, acc):
    cid = lax.axis_index("c")
    sid = lax.axis_index("s")
    wid = sid * NC + cid
    epw = row_hbm.shape[0] // NW

    @pl.when(sid == 0)
    def _():
        pltpu.sync_copy(zeros_hbm, acc)

    pltpu.sync_copy(u_hbm, u_local)
    pltpu.sync_copy(row_hbm.at[pl.ds(wid * epw, epw)], rowbuf)
    pltpu.sync_copy(col_hbm.at[pl.ds(wid * epw, epw)], colbuf)

    def body(i, carry):
        idx = rowbuf[pl.ds(i * 16, 16)]
        vals[pl.ds(i * 16, 16)] = plsc.load_gather(u_local, [idx])
        return carry

    lax.fori_loop(0, epw // 16, body, 0)
    plsc.subcore_barrier()
    pltpu.sync_copy(vals, acc.at[colbuf], add=True)
    plsc.subcore_barrier()

    @pl.when(sid == 0)
    def _():
        pltpu.sync_copy(acc, out_hbm.at[cid])


def _edge2(row, col, u_flat, zeros_n):
    epw = row.shape[0] // NW
    kfn = pl.kernel(
        _edge2_body,
        out_type=jax.ShapeDtypeStruct((NC, N_NODES), jnp.float32),
        mesh=plsc.VectorSubcoreMesh(core_axis_name="c", subcore_axis_name="s",
                                    num_cores=NC, num_subcores=NS),
        compiler_params=pltpu.CompilerParams(use_tc_tiling_on_sc=False,
                                             needs_layout_passes=False,
                                             has_side_effects=pltpu.SideEffectType.PURE),
        scratch_types=[
            pltpu.VMEM((epw,), jnp.int32),
            pltpu.VMEM((epw,), jnp.int32),
            pltpu.VMEM((epw,), jnp.float32),
            pltpu.VMEM((N_NODES,), jnp.float32),
            pltpu.VMEM_SHARED((N_NODES,), jnp.float32),
        ],
    )
    return kfn(row, col, u_flat, zeros_n)


def _final_body(p2_ref, u_ref, dinv_ref, b2_ref, o_ref):
    seg = p2_ref[0] + p2_ref[1]
    z = dinv_ref[...] * (seg + u_ref[...]) + b2_ref[...]
    m = jnp.max(z, axis=1, keepdims=True)
    lse = m + jnp.log(jnp.sum(jnp.exp(z - m), axis=1, keepdims=True))
    o_ref[...] = z - lse


def _final(p2, u, dinv, b2):
    return pl.pallas_call(
        _final_body,
        out_shape=jax.ShapeDtypeStruct((N_NODES, 1), jnp.float32),
    )(p2, u, dinv, b2)


def kernel(x, edge_index, W1, b1, W2, b2):
    row = edge_index[0].astype(jnp.int32)
    col = edge_index[1].astype(jnp.int32)

    ones_c = jnp.ones((row.shape[0] // NW,), jnp.float32)
    zeros_n = jnp.zeros((N_NODES,), jnp.float32)
    zeros_n16 = jnp.zeros((N_NODES, FP), jnp.float32)
    b1p = jnp.zeros((1, FP), jnp.float32).at[0, :F_OUT].set(b1)
    w2p = jnp.zeros((1, FP), jnp.float32).at[0, :F_OUT].set(W2[:, 0])

    degp = _hist(col, ones_c, zeros_n)
    xw = _matmul(x, W1)
    xw_pad = jnp.pad(xw, ((0, 0), (0, FP - F_OUT)))
    y, dinv = _scale(xw_pad, degp.reshape(NC, N_NODES, 1))
    p1 = _edge1(row, col, y, zeros_n16)
    u = _mid(p1, y, dinv, b1p, w2p)
    p2 = _edge2(row, col, u.reshape(N_NODES), zeros_n)
    out = _final(p2.reshape(NC, N_NODES, 1), u, dinv,
                 b2.reshape(1, 1))
    return out


# speedup vs baseline: 1.0210x; 1.0210x over previous
import functools

import jax
import jax.numpy as jnp
from jax import lax
from jax.experimental import pallas as pl
from jax.experimental.pallas import tpu as pltpu
from jax.experimental.pallas import tpu_sc as plsc

N_NODES = 10000
F_IN = 10000
F_OUT = 5
FP = 8
NC, NS = 2, 16
NW = NC * NS
CHUNK = 1000


def _mm_body(x_ref, w_ref, degp_ref, y_ref, dinv_ref):
    xw = jnp.dot(x_ref[...], w_ref[...],
                 preferred_element_type=jnp.float32)
    deg = degp_ref[0] + degp_ref[1] + 1.0
    dinv = lax.rsqrt(deg)
    dinv_ref[...] = dinv
    y_ref[...] = jnp.concatenate(
        [xw * dinv, jnp.zeros((xw.shape[0], FP - F_OUT), jnp.float32)],
        axis=1)


def _matmul_scale(x, w, degp_r):
    m, k = x.shape
    f = w.shape[1]
    bm = 400
    return pl.pallas_call(
        _mm_body,
        grid=(m // bm,),
        in_specs=[
            pl.BlockSpec((bm, k), lambda i: (i, 0)),
            pl.BlockSpec((k, f), lambda i: (0, 0)),
            pl.BlockSpec((NC, bm, 1), lambda i: (0, i, 0)),
        ],
        out_specs=(
            pl.BlockSpec((bm, FP), lambda i: (i, 0)),
            pl.BlockSpec((bm, 1), lambda i: (i, 0)),
        ),
        out_shape=(
            jax.ShapeDtypeStruct((m, FP), jnp.float32),
            jax.ShapeDtypeStruct((m, 1), jnp.float32),
        ),
    )(x, w, degp_r)


def _hist_body(col_hbm, ones_hbm, zeros_hbm, deg_hbm, colbuf, valbuf, acc):
    cid = lax.axis_index("c")
    sid = lax.axis_index("s")
    wid = sid * NC + cid
    epw = col_hbm.shape[0] // NW

    @pl.when(sid == 0)
    def _():
        pltpu.sync_copy(zeros_hbm, acc)

    pltpu.sync_copy(col_hbm.at[pl.ds(wid * epw, epw)], colbuf)
    pltpu.sync_copy(ones_hbm, valbuf)
    plsc.subcore_barrier()
    pltpu.sync_copy(valbuf, acc.at[colbuf], add=True)
    plsc.subcore_barrier()

    @pl.when(sid == 0)
    def _():
        pltpu.sync_copy(acc, deg_hbm.at[cid])


def _hist(col, ones_c, zeros_n):
    epw = col.shape[0] // NW
    kfn = pl.kernel(
        _hist_body,
        cost_estimate=pl.CostEstimate(flops=col.shape[0], transcendentals=0,
                                      bytes_accessed=8 * col.shape[0]),
        out_type=jax.ShapeDtypeStruct((NC, N_NODES), jnp.float32),
        mesh=plsc.VectorSubcoreMesh(core_axis_name="c", subcore_axis_name="s",
                                    num_cores=NC, num_subcores=NS),
        compiler_params=pltpu.CompilerParams(use_tc_tiling_on_sc=False,
                                             needs_layout_passes=False),
        scratch_types=[
            pltpu.VMEM((epw,), jnp.int32),
            pltpu.VMEM((epw,), jnp.float32),
            pltpu.VMEM_SHARED((N_NODES,), jnp.float32),
        ],
    )
    return kfn(col, ones_c, zeros_n)


def _edge1_body(row_hbm, col_hbm, y_hbm, zeros_hbm, out_hbm,
                rowbuf, colbuf, rows, acc,
                sg0, sg1, sg2, ss0, ss1, ss2):
    cid = lax.axis_index("c")
    sid = lax.axis_index("s")
    wid = sid * NC + cid
    epw = row_hbm.shape[0] // NW
    ck = CHUNK
    nchunks = epw // ck
    sems_g = (sg0, sg1, sg2)
    sems_s = (ss0, ss1, ss2)

    @pl.when(sid == 0)
    def _():
        pltpu.sync_copy(zeros_hbm, acc)

    plsc.subcore_barrier()

    def load_idx(j, b):
        base = wid * epw + j * ck
        pltpu.sync_copy(row_hbm.at[pl.ds(base, ck)], rowbuf.at[b])
        pltpu.sync_copy(col_hbm.at[pl.ds(base, ck)], colbuf.at[b])

    gathers = []
    for jj in range(min(2, nchunks)):
        load_idx(jj, jj)
        gathers.append(pltpu.async_copy(y_hbm.at[rowbuf.at[jj]],
                                        rows.at[jj], sems_g[jj]))
    scats = []
    for j in range(nchunks):
        b = j % 3
        gathers[j].wait()
        scats.append(pltpu.async_copy(rows.at[b], acc.at[colbuf.at[b]],
                                      sems_s[b], add=True))
        nxt = j + 2
        if nxt < nchunks:
            nb = nxt % 3
            if nxt - 3 >= 0:
                scats[nxt - 3].wait()
            load_idx(nxt, nb)
            gathers.append(pltpu.async_copy(y_hbm.at[rowbuf.at[nb]],
                                            rows.at[nb], sems_g[nb]))
    for j in range(max(0, nchunks - 3), nchunks):
        scats[j].wait()
    plsc.subcore_barrier()

    @pl.when(sid == 0)
    def _():
        pltpu.sync_copy(acc, out_hbm.at[cid])


def _edge1(row, col, y, zeros_n16):
    kfn = pl.kernel(
        _edge1_body,
        out_type=jax.ShapeDtypeStruct((NC, N_NODES, FP), jnp.float32),
        mesh=plsc.VectorSubcoreMesh(core_axis_name="c", subcore_axis_name="s",
                                    num_cores=NC, num_subcores=NS),
        compiler_params=pltpu.CompilerParams(use_tc_tiling_on_sc=False,
                                             needs_layout_passes=False),
        scratch_types=[
            pltpu.VMEM((3, CHUNK), jnp.int32),
            pltpu.VMEM((3, CHUNK), jnp.int32),
            pltpu.VMEM((3, CHUNK, FP), jnp.float32),
            pltpu.VMEM_SHARED((N_NODES, FP), jnp.float32),
            pltpu.SemaphoreType.DMA,
            pltpu.SemaphoreType.DMA,
            pltpu.SemaphoreType.DMA,
            pltpu.SemaphoreType.DMA,
            pltpu.SemaphoreType.DMA,
            pltpu.SemaphoreType.DMA,
        ],
    )
    return kfn(row, col, y, zeros_n16)


def _mid_body(p1_ref, y_ref, dinv_ref, b1p_ref, w2p_ref, u_ref):
    s = p1_ref[0] + p1_ref[1] + y_ref[...]
    dinv = dinv_ref[...]
    h = jnp.maximum(s * dinv + b1p_ref[...], 0.0)
    xw2 = jnp.sum(h * w2p_ref[...], axis=1, keepdims=True)
    u_ref[...] = xw2 * dinv


def _mid(p1, y, dinv, b1p, w2p):
    return pl.pallas_call(
        _mid_body,
        out_shape=jax.ShapeDtypeStruct((N_NODES, 1), jnp.float32),
    )(p1, y, dinv, b1p, w2p)


def _edge2_body(row_hbm, col_hbm, u_hbm, zeros_hbm, out_hbm,
                rowbuf, colbuf, vals, u_local, acc):
    cid = lax.axis_index("c")
    sid = lax.axis_index("s")
    wid = sid * NC + cid
    epw = row_hbm.shape[0] // NW

    @pl.when(sid == 0)
    def _():
        pltpu.sync_copy(zeros_hbm, acc)

    pltpu.sync_copy(u_hbm, u_local)
    pltpu.sync_copy(row_hbm.at[pl.ds(wid * epw, epw)], rowbuf)
    pltpu.sync_copy(col_hbm.at[pl.ds(wid * epw, epw)], colbuf)

    def body(i, carry):
        idx = rowbuf[pl.ds(i * 16, 16)]
        vals[pl.ds(i * 16, 16)] = plsc.load_gather(u_local, [idx])
        return carry

    lax.fori_loop(0, epw // 16, body, 0)
    plsc.subcore_barrier()
    pltpu.sync_copy(vals, acc.at[colbuf], add=True)
    plsc.subcore_barrier()

    @pl.when(sid == 0)
    def _():
        pltpu.sync_copy(acc, out_hbm.at[cid])


def _edge2(row, col, u_flat, zeros_n):
    epw = row.shape[0] // NW
    kfn = pl.kernel(
        _edge2_body,
        out_type=jax.ShapeDtypeStruct((NC, N_NODES), jnp.float32),
        mesh=plsc.VectorSubcoreMesh(core_axis_name="c", subcore_axis_name="s",
                                    num_cores=NC, num_subcores=NS),
        compiler_params=pltpu.CompilerParams(use_tc_tiling_on_sc=False,
                                             needs_layout_passes=False),
        scratch_types=[
            pltpu.VMEM((epw,), jnp.int32),
            pltpu.VMEM((epw,), jnp.int32),
            pltpu.VMEM((epw,), jnp.float32),
            pltpu.VMEM((N_NODES,), jnp.float32),
            pltpu.VMEM_SHARED((N_NODES,), jnp.float32),
        ],
    )
    return kfn(row, col, u_flat, zeros_n)


def _final_body(p2_ref, u_ref, dinv_ref, b2_ref, o_ref):
    seg = p2_ref[0] + p2_ref[1]
    z = dinv_ref[...] * (seg + u_ref[...]) + b2_ref[...]
    m = jnp.max(z, axis=1, keepdims=True)
    lse = m + jnp.log(jnp.sum(jnp.exp(z - m), axis=1, keepdims=True))
    o_ref[...] = z - lse


def _final(p2, u, dinv, b2):
    return pl.pallas_call(
        _final_body,
        out_shape=jax.ShapeDtypeStruct((N_NODES, 1), jnp.float32),
    )(p2, u, dinv, b2)


def kernel(x, edge_index, W1, b1, W2, b2):
    row = edge_index[0].astype(jnp.int32)
    col = edge_index[1].astype(jnp.int32)

    ones_c = jnp.ones((row.shape[0] // NW,), jnp.float32)
    zeros_n = jnp.zeros((N_NODES,), jnp.float32)
    zeros_n16 = jnp.zeros((N_NODES, FP), jnp.float32)
    b1p = jnp.zeros((1, FP), jnp.float32).at[0, :F_OUT].set(b1)
    w2p = jnp.zeros((1, FP), jnp.float32).at[0, :F_OUT].set(W2[:, 0])

    degp = _hist(col, ones_c, zeros_n)
    y, dinv = _matmul_scale(x, W1,
                            degp.reshape(NC, N_NODES, 1))
    p1 = _edge1(row, col, y, zeros_n16)
    u = _mid(p1, y, dinv, b1p, w2p)
    p2 = _edge2(row, col, u.reshape(N_NODES), zeros_n)
    out = _final(p2.reshape(NC, N_NODES, 1), u, dinv,
                 b2.reshape(1, 1))
    return out


# speedup vs baseline: 1.0253x; 1.0042x over previous
import jax
import jax.numpy as jnp
from jax import lax
from jax.experimental import pallas as pl
from jax.experimental.pallas import tpu as pltpu
from jax.experimental.pallas import tpu_sc as plsc

N_NODES = 10000
F_IN = 10000
F_OUT = 5
FP = 8
NC, NS = 2, 16
NW = NC * NS
CHUNK = 2000


def _mm_body(x_ref, w_ref, degp_ref, y_ref, dinv_ref):
    xw = jnp.dot(x_ref[...], w_ref[...],
                 preferred_element_type=jnp.float32)
    deg = degp_ref[0] + degp_ref[1] + 1.0
    dinv = lax.rsqrt(deg)
    dinv_ref[...] = dinv
    y_ref[...] = jnp.concatenate(
        [xw * dinv, jnp.zeros((xw.shape[0], FP - F_OUT), jnp.float32)],
        axis=1)


def _matmul_scale(x, w, degp_r):
    m, k = x.shape
    f = w.shape[1]
    bm = 400
    return pl.pallas_call(
        _mm_body,
        grid=(m // bm,),
        in_specs=[
            pl.BlockSpec((bm, k), lambda i: (i, 0)),
            pl.BlockSpec((k, f), lambda i: (0, 0)),
            pl.BlockSpec((NC, bm, 1), lambda i: (0, i, 0)),
        ],
        out_specs=(
            pl.BlockSpec((bm, FP), lambda i: (i, 0)),
            pl.BlockSpec((bm, 1), lambda i: (i, 0)),
        ),
        out_shape=(
            jax.ShapeDtypeStruct((m, FP), jnp.float32),
            jax.ShapeDtypeStruct((m, 1), jnp.float32),
        ),
    )(x, w, degp_r)


def _hist_body(col_hbm, ones_hbm, zeros_hbm, deg_hbm, colbuf, valbuf, acc):
    cid = lax.axis_index("c")
    sid = lax.axis_index("s")
    wid = sid * NC + cid
    epw = col_hbm.shape[0] // NW

    @pl.when(sid == 0)
    def _():
        pltpu.sync_copy(zeros_hbm, acc)

    pltpu.sync_copy(col_hbm.at[pl.ds(wid * epw, epw)], colbuf)
    pltpu.sync_copy(ones_hbm, valbuf)
    plsc.subcore_barrier()
    pltpu.sync_copy(valbuf, acc.at[colbuf], add=True)
    plsc.subcore_barrier()

    @pl.when(sid == 0)
    def _():
        pltpu.sync_copy(acc, deg_hbm.at[cid])


def _hist(col, ones_c, zeros_n):
    epw = col.shape[0] // NW
    kfn = pl.kernel(
        _hist_body,
        out_type=jax.ShapeDtypeStruct((NC, N_NODES), jnp.float32),
        mesh=plsc.VectorSubcoreMesh(core_axis_name="c", subcore_axis_name="s",
                                    num_cores=NC, num_subcores=NS),
        compiler_params=pltpu.CompilerParams(use_tc_tiling_on_sc=False,
                                             needs_layout_passes=False),
        scratch_types=[
            pltpu.VMEM((epw,), jnp.int32),
            pltpu.VMEM((epw,), jnp.float32),
            pltpu.VMEM_SHARED((N_NODES,), jnp.float32),
        ],
    )
    return kfn(col, ones_c, zeros_n)


def _edge1_body(row_hbm, col_hbm, y_hbm, zeros_hbm, out_hbm,
                rowbuf, colbuf, rows, acc, sem0, sem1):
    cid = lax.axis_index("c")
    sid = lax.axis_index("s")
    wid = sid * NC + cid
    epw = row_hbm.shape[0] // NW
    nchunks = epw // CHUNK
    sems = (sem0, sem1)

    @pl.when(sid == 0)
    def _():
        pltpu.sync_copy(zeros_hbm, acc)

    plsc.subcore_barrier()

    def load_idx(j, b):
        base = wid * epw + j * CHUNK
        pltpu.sync_copy(row_hbm.at[pl.ds(base, CHUNK)], rowbuf.at[b])
        pltpu.sync_copy(col_hbm.at[pl.ds(base, CHUNK)], colbuf.at[b])

    load_idx(0, 0)
    gathers = [pltpu.async_copy(y_hbm.at[rowbuf.at[0]], rows.at[0], sems[0])]
    for j in range(nchunks):
        b = j % 2
        nb = (j + 1) % 2
        if j + 1 < nchunks:
            load_idx(j + 1, nb)
            gathers.append(
                pltpu.async_copy(y_hbm.at[rowbuf.at[nb]], rows.at[nb],
                                 sems[nb]))
        gathers[j].wait()
        pltpu.sync_copy(rows.at[b], acc.at[colbuf.at[b]], add=True)
    plsc.subcore_barrier()

    @pl.when(sid == 0)
    def _():
        pltpu.sync_copy(acc, out_hbm.at[cid])


def _edge1(row, col, y, zeros_n16):
    kfn = pl.kernel(
        _edge1_body,
        out_type=jax.ShapeDtypeStruct((NC, N_NODES, FP), jnp.float32),
        mesh=plsc.VectorSubcoreMesh(core_axis_name="c", subcore_axis_name="s",
                                    num_cores=NC, num_subcores=NS),
        compiler_params=pltpu.CompilerParams(use_tc_tiling_on_sc=False,
                                             needs_layout_passes=False),
        scratch_types=[
            pltpu.VMEM((2, CHUNK), jnp.int32),
            pltpu.VMEM((2, CHUNK), jnp.int32),
            pltpu.VMEM((2, CHUNK, FP), jnp.float32),
            pltpu.VMEM_SHARED((N_NODES, FP), jnp.float32),
            pltpu.SemaphoreType.DMA,
            pltpu.SemaphoreType.DMA,
        ],
    )
    return kfn(row, col, y, zeros_n16)


def _mid_body(p1_ref, y_ref, dinv_ref, b1p_ref, w2p_ref, u_ref):
    s = p1_ref[0] + p1_ref[1] + y_ref[...]
    dinv = dinv_ref[...]
    h = jnp.maximum(s * dinv + b1p_ref[...], 0.0)
    xw2 = jnp.sum(h * w2p_ref[...], axis=1, keepdims=True)
    u_ref[...] = xw2 * dinv


def _mid(p1, y, dinv, b1p, w2p):
    return pl.pallas_call(
        _mid_body,
        out_shape=jax.ShapeDtypeStruct((N_NODES, 1), jnp.float32),
    )(p1, y, dinv, b1p, w2p)


def _edge2_body(row_hbm, col_hbm, u_hbm, zeros_hbm, out_hbm,
                rowbuf, colbuf, vals, u_local, acc):
    cid = lax.axis_index("c")
    sid = lax.axis_index("s")
    wid = sid * NC + cid
    epw = row_hbm.shape[0] // NW

    @pl.when(sid == 0)
    def _():
        pltpu.sync_copy(zeros_hbm, acc)

    pltpu.sync_copy(u_hbm, u_local)
    pltpu.sync_copy(row_hbm.at[pl.ds(wid * epw, epw)], rowbuf)
    pltpu.sync_copy(col_hbm.at[pl.ds(wid * epw, epw)], colbuf)

    def body(i, carry):
        idx = rowbuf[pl.ds(i * 16, 16)]
        vals[pl.ds(i * 16, 16)] = plsc.load_gather(u_local, [idx])
        return carry

    lax.fori_loop(0, epw // 16, body, 0)
    plsc.subcore_barrier()
    pltpu.sync_copy(vals, acc.at[colbuf], add=True)
    plsc.subcore_barrier()

    @pl.when(sid == 0)
    def _():
        pltpu.sync_copy(acc, out_hbm.at[cid])


def _edge2(row, col, u_flat, zeros_n):
    epw = row.shape[0] // NW
    kfn = pl.kernel(
        _edge2_body,
        out_type=jax.ShapeDtypeStruct((NC, N_NODES), jnp.float32),
        mesh=plsc.VectorSubcoreMesh(core_axis_name="c", subcore_axis_name="s",
                                    num_cores=NC, num_subcores=NS),
        compiler_params=pltpu.CompilerParams(use_tc_tiling_on_sc=False,
                                             needs_layout_passes=False),
        scratch_types=[
            pltpu.VMEM((epw,), jnp.int32),
            pltpu.VMEM((epw,), jnp.int32),
            pltpu.VMEM((epw,), jnp.float32),
            pltpu.VMEM((N_NODES,), jnp.float32),
            pltpu.VMEM_SHARED((N_NODES,), jnp.float32),
        ],
    )
    return kfn(row, col, u_flat, zeros_n)


def _final_body(p2_ref, u_ref, dinv_ref, b2_ref, o_ref):
    seg = p2_ref[0] + p2_ref[1]
    z = dinv_ref[...] * (seg + u_ref[...]) + b2_ref[...]
    m = jnp.max(z, axis=1, keepdims=True)
    lse = m + jnp.log(jnp.sum(jnp.exp(z - m), axis=1, keepdims=True))
    o_ref[...] = z - lse


def _final(p2, u, dinv, b2):
    return pl.pallas_call(
        _final_body,
        out_shape=jax.ShapeDtypeStruct((N_NODES, 1), jnp.float32),
    )(p2, u, dinv, b2)


def kernel(x, edge_index, W1, b1, W2, b2):
    row = edge_index[0].astype(jnp.int32)
    col = edge_index[1].astype(jnp.int32)

    ones_c = jnp.ones((row.shape[0] // NW,), jnp.float32)
    zeros_n = jnp.zeros((N_NODES,), jnp.float32)
    zeros_n16 = jnp.zeros((N_NODES, FP), jnp.float32)
    b1p = jnp.zeros((1, FP), jnp.float32).at[0, :F_OUT].set(b1)
    w2p = jnp.zeros((1, FP), jnp.float32).at[0, :F_OUT].set(W2[:, 0])

    degp = _hist(col, ones_c, zeros_n)
    y, dinv = _matmul_scale(x, W1,
                            degp.reshape(NC, N_NODES, 1))
    p1 = _edge1(row, col, y, zeros_n16)
    u = _mid(p1, y, dinv, b1p, w2p)
    p2 = _edge2(row, col, u.reshape(N_NODES), zeros_n)
    out = _final(p2.reshape(NC, N_NODES, 1), u, dinv,
                 b2.reshape(1, 1))
    return out
